# Initial kernel scaffold; baseline (speedup 1.0000x reference)
import functools

import jax
import jax.numpy as jnp
from jax import lax
from jax.experimental import pallas as pl
from jax.experimental.pallas import tpu as pltpu
from jax.experimental.pallas import tpu_sc as plsc

N_USERS = 5000
N_ITEMS = 5000
N = N_USERS + N_ITEMS
E = 320000
D = 128
L = 2

NS = 16
C = 80
EPT = E // NS
RPT = N // NS


def _sc_aggregate(ego_cat, src_cat, dst_cat, ew_cat, zeros_nd):
    mesh = plsc.VectorSubcoreMesh(core_axis_name="c", subcore_axis_name="s")

    @functools.partial(
        pl.kernel,
        out_type=jax.ShapeDtypeStruct((2 * N, D), jnp.float32),
        mesh=mesh,
        scratch_types=[
            pltpu.VMEM((C,), jnp.int32),
            pltpu.VMEM((C,), jnp.int32),
            pltpu.VMEM((C,), jnp.float32),
            pltpu.VMEM((C, D), jnp.float32),
            pltpu.VMEM_SHARED((N, D), jnp.float32),
            pltpu.SemaphoreType.DMA,
        ],
    )
    def agg(ego_hbm, src_hbm, dst_hbm, ew_hbm, zero_hbm, out_hbm,
            idx_src, idx_dst, ew_v, rows, acc, sem):
        cid = lax.axis_index("c")
        sid = lax.axis_index("s")
        pltpu.sync_copy(zero_hbm.at[pl.ds(sid * RPT, RPT)],
                        acc.at[pl.ds(sid * RPT, RPT)])
        plsc.subcore_barrier()

        row_base = cid * N
        ebase = cid * E + sid * EPT

        def chunk(i, carry):
            off = ebase + i * C
            pltpu.sync_copy(src_hbm.at[pl.ds(off, C)], idx_src)
            pltpu.sync_copy(dst_hbm.at[pl.ds(off, C)], idx_dst)
            pltpu.sync_copy(ew_hbm.at[pl.ds(off, C)], ew_v)
            for j in range(C // 16):
                idx_src[pl.ds(j * 16, 16)] = idx_src[pl.ds(j * 16, 16)] + row_base
            pltpu.async_copy(ego_hbm.at[idx_src], rows, sem).wait()

            def rowfn(r, c2):
                w = ew_v[r]
                for j in range(D // 16):
                    rows[r, pl.ds(j * 16, 16)] = rows[r, pl.ds(j * 16, 16)] * w
                return c2

            lax.fori_loop(0, C, rowfn, 0)
            pltpu.sync_copy(rows, acc.at[idx_dst], add=True)
            return carry

        lax.fori_loop(0, EPT // C, chunk, 0)
        plsc.subcore_barrier()
        pltpu.sync_copy(acc.at[pl.ds(sid * RPT, RPT)],
                        out_hbm.at[pl.ds(cid * N + sid * RPT, RPT)])

    return agg(ego_cat, src_cat, dst_cat, ew_cat, zeros_nd)


def _ew_body(side_ref, ego_ref, newego_ref, norm_ref):
    s = side_ref[...]
    e = ego_ref[...]
    ne = s + e * s
    newego_ref[...] = ne
    ss = jnp.sum(ne * ne, axis=1, keepdims=True)
    denom = jnp.maximum(jnp.sqrt(ss), 1e-12)
    norm_ref[...] = ne / denom


def _tc_layer(side, ego):
    br = 1000
    grid = (2 * N // br,)
    return pl.pallas_call(
        _ew_body,
        grid=grid,
        in_specs=[pl.BlockSpec((br, D), lambda i: (i, 0)),
                  pl.BlockSpec((br, D), lambda i: (i, 0))],
        out_specs=[pl.BlockSpec((br, D), lambda i: (i, 0)),
                   pl.BlockSpec((br, D), lambda i: (i, 0))],
        out_shape=[jax.ShapeDtypeStruct((2 * N, D), jnp.float32),
                   jax.ShapeDtypeStruct((2 * N, D), jnp.float32)],
    )(side, ego)


def _proj_body(x_ref, w_ref, b_ref, o_ref):
    y = jnp.dot(x_ref[...], w_ref[0], preferred_element_type=jnp.float32)
    y = y + b_ref[0]
    o_ref[...] = jnp.where(y >= 0, y, 0.01 * y)


def _tc_project(allx, w4, b4):
    br = 1000
    nb = N_USERS // br
    grid = (4, nb)
    return pl.pallas_call(
        _proj_body,
        grid=grid,
        in_specs=[
            pl.BlockSpec((br, (L + 1) * D), lambda i, j: (i * nb + j, 0)),
            pl.BlockSpec((1, (L + 1) * D, D), lambda i, j: (i, 0, 0)),
            pl.BlockSpec((1, D), lambda i, j: (i, 0)),
        ],
        out_specs=pl.BlockSpec((br, D), lambda i, j: (i * nb + j, 0)),
        out_shape=jax.ShapeDtypeStruct((4 * N_USERS, D), jnp.float32),
    )(allx, w4, b4)


def kernel(user_emb_s, item_emb_s, user_emb_t, item_emb_t, ew_s, ew_t,
           W_s, b_s, W_si, b_si, W_t, b_t, W_ti, b_ti,
           src_s, dst_s, src_t, dst_t):
    ego = jnp.concatenate([user_emb_s, item_emb_s,
                           user_emb_t, item_emb_t], axis=0)
    src = jnp.concatenate([src_s, src_t])
    dst = jnp.concatenate([dst_s, dst_t])
    ew = jnp.concatenate([ew_s, ew_t])
    zeros_nd = jnp.zeros((N, D), jnp.float32)

    embs = [ego]
    for _ in range(L):
        side = _sc_aggregate(ego, src, dst, ew, zeros_nd)
        ego, nrm = _tc_layer(side, ego)
        embs.append(nrm)

    allx = jnp.concatenate(embs, axis=1)
    w4 = jnp.stack([W_s, W_si, W_t, W_ti])
    b4 = jnp.stack([b_s, b_si, b_t, b_ti])
    return _tc_project(allx, w4, b4)


# speedup vs baseline: 2.7379x; 2.7379x over previous
import functools

import jax
import jax.numpy as jnp
from jax import lax
from jax.experimental import pallas as pl
from jax.experimental.pallas import tpu as pltpu
from jax.experimental.pallas import tpu_sc as plsc

N_USERS = 5000
N_ITEMS = 5000
N = N_USERS + N_ITEMS
E = 320000
D = 128
L = 2

NS = 16
C = 80
EPT = E // NS
NPAD = 10240
RPT = NPAD // NS
RLAST = N - (NS - 1) * RPT


def _sc_aggregate(ego_cat, src_cat, dst_cat, ew_cat, zeros_nd):
    mesh = plsc.VectorSubcoreMesh(core_axis_name="c", subcore_axis_name="s",
                                  num_cores=2, num_subcores=NS)

    @functools.partial(
        pl.kernel,
        out_type=jax.ShapeDtypeStruct((2 * N, D), jnp.float32),
        mesh=mesh,
        scratch_types=[
            pltpu.VMEM((C,), jnp.int32),
            pltpu.VMEM((C,), jnp.int32),
            pltpu.VMEM((C,), jnp.float32),
            pltpu.VMEM((C, D), jnp.float32),
            pltpu.VMEM_SHARED((NPAD, D), jnp.float32),
            pltpu.SemaphoreType.DMA,
        ],
    )
    def agg(ego_hbm, src_hbm, dst_hbm, ew_hbm, zero_hbm, out_hbm,
            idx_src, idx_dst, ew_v, rows, acc, sem):
        cid = lax.axis_index("c")
        sid = lax.axis_index("s")
        pltpu.sync_copy(zero_hbm.at[pl.ds(sid * RPT, RPT)],
                        acc.at[pl.ds(sid * RPT, RPT)])
        plsc.subcore_barrier()

        row_base = cid * N
        ebase = cid * E + sid * EPT

        def chunk(i, carry):
            off = ebase + i * C
            pltpu.sync_copy(src_hbm.at[pl.ds(off, C)], idx_src)
            pltpu.sync_copy(dst_hbm.at[pl.ds(off, C)], idx_dst)
            pltpu.sync_copy(ew_hbm.at[pl.ds(off, C)], ew_v)
            for j in range(C // 16):
                idx_src[pl.ds(j * 16, 16)] = idx_src[pl.ds(j * 16, 16)] + row_base
            pltpu.async_copy(ego_hbm.at[idx_src], rows, sem).wait()

            def grpfn(g, c2):
                w16 = ew_v[pl.ds(g * 16, 16)]
                for r in range(16):
                    w = w16[r]
                    rr = g * 16 + r
                    for j in range(D // 16):
                        rows[rr, pl.ds(j * 16, 16)] = rows[rr, pl.ds(j * 16, 16)] * w
                return c2

            lax.fori_loop(0, C // 16, grpfn, 0)
            pltpu.sync_copy(rows, acc.at[idx_dst], add=True)
            return carry

        lax.fori_loop(0, EPT // C, chunk, 0)
        plsc.subcore_barrier()

        @pl.when(sid < NS - 1)
        def _():
            pltpu.sync_copy(acc.at[pl.ds(sid * RPT, RPT)],
                            out_hbm.at[pl.ds(cid * N + sid * RPT, RPT)])

        @pl.when(sid == NS - 1)
        def _():
            pltpu.sync_copy(acc.at[pl.ds((NS - 1) * RPT, RLAST)],
                            out_hbm.at[pl.ds(cid * N + (NS - 1) * RPT, RLAST)])

    return agg(ego_cat, src_cat, dst_cat, ew_cat, zeros_nd)


def _ew_body(side_ref, ego_ref, newego_ref, norm_ref):
    s = side_ref[...]
    e = ego_ref[...]
    ne = s + e * s
    newego_ref[...] = ne
    ss = jnp.sum(ne * ne, axis=1, keepdims=True)
    denom = jnp.maximum(jnp.sqrt(ss), 1e-12)
    norm_ref[...] = ne / denom


def _tc_layer(side, ego):
    br = 1000
    grid = (2 * N // br,)
    return pl.pallas_call(
        _ew_body,
        grid=grid,
        in_specs=[pl.BlockSpec((br, D), lambda i: (i, 0)),
                  pl.BlockSpec((br, D), lambda i: (i, 0))],
        out_specs=[pl.BlockSpec((br, D), lambda i: (i, 0)),
                   pl.BlockSpec((br, D), lambda i: (i, 0))],
        out_shape=[jax.ShapeDtypeStruct((2 * N, D), jnp.float32),
                   jax.ShapeDtypeStruct((2 * N, D), jnp.float32)],
    )(side, ego)


def _proj_body(x_ref, w_ref, b_ref, o_ref):
    y = jnp.dot(x_ref[...], w_ref[0], preferred_element_type=jnp.float32)
    y = y + b_ref[0, 0]
    o_ref[...] = jnp.where(y >= 0, y, 0.01 * y)


def _tc_project(allx, w4, b4):
    br = 1000
    nb = N_USERS // br
    grid = (4, nb)
    return pl.pallas_call(
        _proj_body,
        grid=grid,
        in_specs=[
            pl.BlockSpec((br, (L + 1) * D), lambda i, j: (i * nb + j, 0)),
            pl.BlockSpec((1, (L + 1) * D, D), lambda i, j: (i, 0, 0)),
            pl.BlockSpec((1, 1, D), lambda i, j: (i, 0, 0)),
        ],
        out_specs=pl.BlockSpec((br, D), lambda i, j: (i * nb + j, 0)),
        out_shape=jax.ShapeDtypeStruct((4 * N_USERS, D), jnp.float32),
    )(allx, w4, b4)


def kernel(user_emb_s, item_emb_s, user_emb_t, item_emb_t, ew_s, ew_t,
           W_s, b_s, W_si, b_si, W_t, b_t, W_ti, b_ti,
           src_s, dst_s, src_t, dst_t):
    ego = jnp.concatenate([user_emb_s, item_emb_s,
                           user_emb_t, item_emb_t], axis=0)
    src = jnp.concatenate([src_s, src_t])
    dst = jnp.concatenate([dst_s, dst_t])
    ew = jnp.concatenate([ew_s, ew_t])
    zeros_nd = jnp.zeros((NPAD, D), jnp.float32)

    embs = [ego]
    for _ in range(L):
        side = _sc_aggregate(ego, src, dst, ew, zeros_nd)
        ego, nrm = _tc_layer(side, ego)
        embs.append(nrm)

    allx = jnp.concatenate(embs, axis=1)
    w4 = jnp.stack([W_s, W_si, W_t, W_ti])
    b4 = jnp.stack([b_s, b_si, b_t, b_ti])[:, None, :]
    return _tc_project(allx, w4, b4)
